# baseline (speedup 1.0000x reference)
import jax
import jax.numpy as jnp
import numpy as np
from jax import lax
from jax.experimental import pallas as pl
from jax.experimental.pallas import tpu as pltpu
from jax.experimental.pallas import tpu_sc as plsc

N_NODES = 100000
MAX_DEGREE = 32
N_IDS = 65536
NUM_SAMPLES = 16

_PERM = np.array(
    [31, 7, 4, 29, 16, 19, 2, 5, 30, 3, 22, 6, 18, 10, 11, 15,
     20, 8, 24, 9, 25, 13, 14, 17, 23, 0, 21, 26, 1, 28, 27, 12],
    dtype=np.int32,
)

_NW = 32
_CHUNK = 3200
_TAIL0 = 31 * _CHUNK
_TAIL1 = 768
_ALIGNED = _TAIL0 + _TAIL1
_BPW = N_IDS // _NW


_UNROLL = 8
_SUBQ = 3


def _k1_body(cols_hbm, adj_hbm, tail_hbm, w_hbm, cols_v, colbuf, wbuf, sems,
             wsems):
    wid = lax.axis_index("s") * 2 + lax.axis_index("c")
    pltpu.sync_copy(cols_hbm, cols_v)
    cvec = cols_v[...]
    lanes = lax.iota(jnp.int32, 16)

    def fire(node0, qoff, nq, slot):
        return [
            pltpu.async_copy(
                adj_hbm.at[cvec, pl.ds(node0 + qoff * 128, nq * 128)],
                colbuf.at[pl.ds(slot * NUM_SAMPLES, NUM_SAMPLES),
                          pl.ds(0, nq * 128)],
                sems[slot],
            )
        ]

    def drain_transpose_write(node0, qoff, nq, slot, copies):
        for cp in copies:
            cp.wait()
        length = nq * 128
        row0 = slot * NUM_SAMPLES
        wb0 = slot * _SUBQ * 128 * NUM_SAMPLES

        @plsc.parallel_loop(0, length, unroll=_UNROLL)
        def step(n):
            vals = plsc.load_gather(
                colbuf,
                [lanes + row0, jnp.full((16,), n, jnp.int32)])
            wbuf[pl.ds(wb0 + n * NUM_SAMPLES, NUM_SAMPLES)] = vals
        return pltpu.async_copy(
            wbuf.at[pl.ds(wb0, length * NUM_SAMPLES)],
            w_hbm.at[pl.ds((node0 + qoff * 128) * NUM_SAMPLES,
                           length * NUM_SAMPLES)],
            wsems[slot])

    def do_subs(node0, subs):
        copies = fire(node0, subs[0][0], subs[0][1], 0)
        wcps = [None, None]
        for i, (qoff, nq) in enumerate(subs):
            nxt = None
            if i + 1 < len(subs):
                nxt = fire(node0, subs[i + 1][0], subs[i + 1][1], (i + 1) % 2)
            if wcps[i % 2] is not None:
                wcps[i % 2].wait()
            wcps[i % 2] = drain_transpose_write(
                node0, qoff, nq, i % 2, copies)
            copies = nxt
        for w in wcps:
            if w is not None:
                w.wait()

    @pl.when(wid < 31)
    def _main():
        do_subs(wid * _CHUNK, [(q, 3) for q in range(0, 24, 3)] + [(24, 1)])

    @pl.when(wid == 31)
    def _tail():
        do_subs(_TAIL0, [(0, _TAIL1 // 128)])
        ntail = (N_NODES - _ALIGNED) * NUM_SAMPLES
        pltpu.sync_copy(tail_hbm, wbuf.at[pl.ds(0, ntail)])
        pltpu.sync_copy(wbuf.at[pl.ds(0, ntail)],
                        w_hbm.at[pl.ds(_ALIGNED * NUM_SAMPLES, ntail)])


_HBPW = _BPW // 2


def _k2_body(ids_hbm, w_hbm, out_hbm, idx_v, rows_v, gsems, wsems):
    wid = lax.axis_index("s") * 2 + lax.axis_index("c")
    base = wid * _BPW
    pltpu.sync_copy(ids_hbm.at[pl.ds(base, _BPW)], idx_v)
    gcps = [
        pltpu.async_copy(w_hbm.at[idx_v.at[pl.ds(h * _HBPW, _HBPW)]],
                         rows_v.at[pl.ds(h * _HBPW, _HBPW), :], gsems[h])
        for h in range(2)
    ]
    wcps = []
    for h in range(2):
        gcps[h].wait()
        wcps.append(pltpu.async_copy(
            rows_v.at[pl.ds(h * _HBPW, _HBPW), :],
            out_hbm.at[pl.ds(base + h * _HBPW, _HBPW)], wsems[h]))
    for cp in wcps:
        cp.wait()


@jax.jit
def _sample(ids, cols, adj_t, adj_info):
    mesh = plsc.VectorSubcoreMesh(core_axis_name="c", subcore_axis_name="s")
    k1 = pl.kernel(
        _k1_body,
        mesh=mesh,
        out_type=jax.ShapeDtypeStruct((N_NODES * NUM_SAMPLES,), jnp.int32),
        scratch_types=[
            pltpu.VMEM((NUM_SAMPLES,), jnp.int32),
            pltpu.VMEM((2 * NUM_SAMPLES, _SUBQ * 128), jnp.int32),
            pltpu.VMEM((2 * _SUBQ * 128 * NUM_SAMPLES,), jnp.int32),
            [pltpu.SemaphoreType.DMA, pltpu.SemaphoreType.DMA],
            [pltpu.SemaphoreType.DMA, pltpu.SemaphoreType.DMA],
        ],
        compiler_params=pltpu.CompilerParams(needs_layout_passes=False),
    )
    tail = jnp.take(adj_info[_ALIGNED:], cols, axis=1).reshape(-1)
    w_flat = k1(cols, adj_t, tail)
    k2 = pl.kernel(
        _k2_body,
        mesh=mesh,
        out_type=jax.ShapeDtypeStruct((N_IDS, NUM_SAMPLES), jnp.int32),
        scratch_types=[
            pltpu.VMEM((_BPW,), jnp.int32),
            pltpu.VMEM((_BPW, NUM_SAMPLES), jnp.int32),
            [pltpu.SemaphoreType.DMA, pltpu.SemaphoreType.DMA],
            [pltpu.SemaphoreType.DMA, pltpu.SemaphoreType.DMA],
        ],
        compiler_params=pltpu.CompilerParams(use_tc_tiling_on_sc=False),
    )
    return k2(ids, w_flat.reshape(N_NODES, NUM_SAMPLES))


def kernel(ids, num_samples, adj_info):
    ids32 = ids.astype(jnp.int32)
    adj32 = adj_info.astype(jnp.int32)
    start = jnp.clip(jnp.asarray(num_samples, jnp.int32) - NUM_SAMPLES,
                     0, MAX_DEGREE - NUM_SAMPLES)
    cols = lax.dynamic_slice(jnp.asarray(_PERM), (start,), (NUM_SAMPLES,))
    out = _sample(ids32, cols, adj32.T, adj32)
    return out.reshape(-1)


# speedup vs baseline: 1.0374x; 1.0374x over previous
import jax
import jax.numpy as jnp
import numpy as np
from jax import lax
from jax.experimental import pallas as pl
from jax.experimental.pallas import tpu as pltpu
from jax.experimental.pallas import tpu_sc as plsc

N_NODES = 100000
MAX_DEGREE = 32
N_IDS = 65536
NUM_SAMPLES = 16

_PERM = np.array(
    [31, 7, 4, 29, 16, 19, 2, 5, 30, 3, 22, 6, 18, 10, 11, 15,
     20, 8, 24, 9, 25, 13, 14, 17, 23, 0, 21, 26, 1, 28, 27, 12],
    dtype=np.int32,
)

_NW = 32
_CHUNK = 3200
_TAIL0 = 31 * _CHUNK
_TAIL1 = 768
_ALIGNED = _TAIL0 + _TAIL1
_BPW = N_IDS // _NW


_UNROLL = 8
_SUBQ = 3


def _k1_body(cols_hbm, adj_hbm, tail_hbm, w_hbm, cols_v, colbuf, wbuf, sems,
             wsems):
    wid = lax.axis_index("s") * 2 + lax.axis_index("c")
    pltpu.sync_copy(cols_hbm, cols_v)
    cvec = cols_v[...]
    lanes = lax.iota(jnp.int32, 16)

    def fire(node0, qoff, nq, slot):
        return [
            pltpu.async_copy(
                adj_hbm.at[cvec, pl.ds(node0 + (qoff + q) * 128, 128)],
                colbuf.at[pl.ds(slot * _SUBQ * NUM_SAMPLES + q * NUM_SAMPLES,
                                NUM_SAMPLES), pl.ds(0, 128)],
                sems[slot],
            )
            for q in range(nq)
        ]

    def drain_transpose_write(node0, qoff, nq, slot, copies):
        for cp in copies:
            cp.wait()
        length = nq * 128
        row0 = slot * _SUBQ * NUM_SAMPLES
        wb0 = slot * _SUBQ * 128 * NUM_SAMPLES

        @plsc.parallel_loop(0, length, unroll=_UNROLL)
        def step(n):
            q = lax.shift_right_logical(n, 7)
            l = jnp.bitwise_and(n, 127)
            vals = plsc.load_gather(
                colbuf,
                [lanes + (row0 + q * NUM_SAMPLES),
                 jnp.full((16,), l, jnp.int32)])
            wbuf[pl.ds(wb0 + n * NUM_SAMPLES, NUM_SAMPLES)] = vals
        return pltpu.async_copy(
            wbuf.at[pl.ds(wb0, length * NUM_SAMPLES)],
            w_hbm.at[pl.ds((node0 + qoff * 128) * NUM_SAMPLES,
                           length * NUM_SAMPLES)],
            wsems[slot])

    def do_subs(node0, subs):
        copies = fire(node0, subs[0][0], subs[0][1], 0)
        wcps = [None, None]
        for i, (qoff, nq) in enumerate(subs):
            nxt = None
            if i + 1 < len(subs):
                nxt = fire(node0, subs[i + 1][0], subs[i + 1][1], (i + 1) % 2)
            if wcps[i % 2] is not None:
                wcps[i % 2].wait()
            wcps[i % 2] = drain_transpose_write(
                node0, qoff, nq, i % 2, copies)
            copies = nxt
        for w in wcps:
            if w is not None:
                w.wait()

    @pl.when(wid < 31)
    def _main():
        do_subs(wid * _CHUNK, [(q, 3) for q in range(0, 24, 3)] + [(24, 1)])

    @pl.when(wid == 31)
    def _tail():
        do_subs(_TAIL0, [(0, _TAIL1 // 128)])
        ntail = (N_NODES - _ALIGNED) * NUM_SAMPLES
        pltpu.sync_copy(tail_hbm, wbuf.at[pl.ds(0, ntail)])
        pltpu.sync_copy(wbuf.at[pl.ds(0, ntail)],
                        w_hbm.at[pl.ds(_ALIGNED * NUM_SAMPLES, ntail)])


_HBPW = _BPW // 4


def _k2_body(ids_hbm, w_hbm, out_hbm, idx_v, rows_v, gsems, wsems):
    wid = lax.axis_index("s") * 2 + lax.axis_index("c")
    base = wid * _BPW
    pltpu.sync_copy(ids_hbm.at[pl.ds(base, _BPW)], idx_v)
    gcps = [
        pltpu.async_copy(w_hbm.at[idx_v.at[pl.ds(h * _HBPW, _HBPW)]],
                         rows_v.at[pl.ds(h * _HBPW, _HBPW), :], gsems[h])
        for h in range(4)
    ]
    wcps = []
    for h in range(4):
        gcps[h].wait()
        wcps.append(pltpu.async_copy(
            rows_v.at[pl.ds(h * _HBPW, _HBPW), :],
            out_hbm.at[pl.ds(base + h * _HBPW, _HBPW)], wsems[h]))
    for cp in wcps:
        cp.wait()


@jax.jit
def _sample(ids, cols, adj_t, adj_info):
    mesh = plsc.VectorSubcoreMesh(core_axis_name="c", subcore_axis_name="s")
    k1 = pl.kernel(
        _k1_body,
        mesh=mesh,
        out_type=jax.ShapeDtypeStruct((N_NODES * NUM_SAMPLES,), jnp.int32),
        scratch_types=[
            pltpu.VMEM((NUM_SAMPLES,), jnp.int32),
            pltpu.VMEM((2 * _SUBQ * NUM_SAMPLES, 137), jnp.int32),
            pltpu.VMEM((2 * _SUBQ * 128 * NUM_SAMPLES,), jnp.int32),
            [pltpu.SemaphoreType.DMA, pltpu.SemaphoreType.DMA],
            [pltpu.SemaphoreType.DMA, pltpu.SemaphoreType.DMA],
        ],
        compiler_params=pltpu.CompilerParams(needs_layout_passes=False),
    )
    tail = jnp.take(adj_info[_ALIGNED:], cols, axis=1).reshape(-1)
    w_flat = k1(cols, adj_t, tail)
    k2 = pl.kernel(
        _k2_body,
        mesh=mesh,
        out_type=jax.ShapeDtypeStruct((N_IDS, NUM_SAMPLES), jnp.int32),
        scratch_types=[
            pltpu.VMEM((_BPW,), jnp.int32),
            pltpu.VMEM((_BPW, NUM_SAMPLES), jnp.int32),
            [pltpu.SemaphoreType.DMA] * 4,
            [pltpu.SemaphoreType.DMA] * 4,
        ],
        compiler_params=pltpu.CompilerParams(use_tc_tiling_on_sc=False),
    )
    return k2(ids, w_flat.reshape(N_NODES, NUM_SAMPLES))


def kernel(ids, num_samples, adj_info):
    ids32 = ids.astype(jnp.int32)
    adj32 = adj_info.astype(jnp.int32)
    start = jnp.clip(jnp.asarray(num_samples, jnp.int32) - NUM_SAMPLES,
                     0, MAX_DEGREE - NUM_SAMPLES)
    cols = lax.dynamic_slice(jnp.asarray(_PERM), (start,), (NUM_SAMPLES,))
    out = _sample(ids32, cols, adj32.T, adj32)
    return out.reshape(-1)
